# baseline (speedup 1.0000x reference)
import functools

import jax
import jax.numpy as jnp
from jax import lax
from jax.experimental import pallas as pl
from jax.experimental.pallas import tpu as pltpu
from jax.experimental.pallas import tpu_sc as plsc

D = 64
NC = 2
NS = 16
NW = NC * NS
BB = 128
NBUF = 4


@functools.cache
def _build_pre(vocab: int):
  assert vocab == 1000000
  nwin = vocab // BB + 1
  tail = nwin - 1
  nstep = 248
  nb = 4
  mesh = plsc.VectorSubcoreMesh(core_axis_name="c", subcore_axis_name="s")

  @functools.partial(
      pl.kernel,
      out_type=jax.ShapeDtypeStruct((vocab // 2, 2 * D), jnp.float32),
      mesh=mesh,
      scratch_types=(
          [pltpu.VMEM((D, BB), jnp.float32) for _ in range(nb)]
          + [pltpu.VMEM((D, BB), jnp.float32) for _ in range(nb)]
          + [pltpu.SemaphoreType.DMA for _ in range(2 * nb)]
      ),
      compiler_params=pltpu.CompilerParams(needs_layout_passes=False),
  )
  def pre_kernel(tt_hbm, tail_hbm, out_hbm, *bufs):
    tin = bufs[:nb]
    tout = bufs[nb:2 * nb]
    si = bufs[2 * nb:3 * nb]
    so = bufs[3 * nb:]
    w = lax.axis_index("s") * NC + lax.axis_index("c")

    lanes = jnp.arange(16, dtype=jnp.int32)
    l2 = 2 * lanes
    kvecs = [lanes + kk0 for kk0 in range(0, D, 16)]

    def win_of(i):
      raw = w + NW * i
      return jnp.where(raw < nwin, raw, 0)

    def load_in(i, s):
      win = win_of(i)

      @pl.when(win == tail)
      def _():
        pltpu.async_copy(tail_hbm, tin[s], si[s])

      @pl.when(win != tail)
      def _():
        pltpu.async_copy(tt_hbm.at[:, pl.ds(win * BB, BB)], tin[s], si[s])

    def wait_in(s):
      pltpu.make_async_copy(tt_hbm.at[:, pl.ds(0, BB)], tin[s], si[s]).wait()

    def store_out(i, s):
      win = win_of(i)

      @pl.when(win == tail)
      def _():
        dst = out_hbm.at[pl.ds(win * D, D // 2)]
        half = tout[s].at[pl.ds(0, D // 2)]
        pltpu.async_copy(half, dst, so[s])
        pltpu.async_copy(half, dst, so[s])

      @pl.when(win != tail)
      def _():
        pltpu.async_copy(tout[s], out_hbm.at[pl.ds(win * D, D)], so[s])

    def wait_out(s):
      pltpu.make_async_copy(
          tout[s], out_hbm.at[pl.ds(0, D)], so[s]).wait()

    for s in range(nb):
      load_in(s, s)

    def step_body(q, carry):
      for s in range(nb):
        i = nb * q + s
        wait_in(s)

        @pl.when(q >= 1)
        def _():
          wait_out(s)

        @plsc.parallel_loop(0, 16, unroll=2)
        def _tr(r):
          rot = (lanes + r) & 15
          for c0 in range(0, 2 * D, 16):
            p = c0 // D
            dvec = rot + (c0 % D)
            cvec = rot + c0
            for ki, kk0 in enumerate(range(0, D, 16)):
              vvec = l2 + (2 * kk0 + p)
              v = plsc.load_gather(tin[s], [dvec, vvec])
              plsc.store_scatter(tout[s], [kvecs[ki], cvec], v)

        store_out(i, s)

        @pl.when(q < nstep // nb - 1)
        def _():
          load_in(i + nb, s)
      return carry

    lax.fori_loop(0, nstep // nb, step_body, 0)
    for s in range(nb):
      wait_out(s)

  return pre_kernel


@functools.cache
def _build(batch: int, hist: int):
  assert batch == NW * BB and hist % NBUF == 0
  nblk = hist // NBUF
  mesh = plsc.VectorSubcoreMesh(core_axis_name="c", subcore_axis_name="s")

  @functools.partial(
      pl.kernel,
      out_type=jax.ShapeDtypeStruct((hist, D // 8, batch // BB, 8, BB),
                                    jnp.float32),
      mesh=mesh,
      scratch_types=[
          pltpu.VMEM((hist, BB), jnp.int32),
      ] + [pltpu.VMEM((BB, 2 * D), jnp.float32) for _ in range(NBUF)]
        + [pltpu.VMEM((D // 8, 8, BB), jnp.float32) for _ in range(2)]
        + [pltpu.VMEM((BB,), jnp.int32) for _ in range(NBUF)]
        + [pltpu.VMEM((BB,), jnp.int32) for _ in range(NBUF)]
        + [pltpu.SemaphoreType.DMA for _ in range(NBUF + 2)],
      compiler_params=pltpu.CompilerParams(needs_layout_passes=False),
  )
  def gather_kernel(tid_hbm, table_hbm, out_hbm, idx_all, *bufs):
    rows = bufs[:NBUF]
    tbuf = bufs[NBUF:NBUF + 2]
    idxp = bufs[NBUF + 2:2 * NBUF + 2]
    parb = bufs[2 * NBUF + 2:3 * NBUF + 2]
    sg = bufs[3 * NBUF + 2:4 * NBUF + 2]
    ss = bufs[4 * NBUF + 2:]
    w = lax.axis_index("s") * NC + lax.axis_index("c")

    pltpu.sync_copy(tid_hbm.at[:, pl.ds(w * BB, BB)], idx_all)

    lanes = jnp.arange(16, dtype=jnp.int32)
    bvecs = [lanes + 16 * j for j in range(BB // 16)]

    def prep_indices(hh, s):
      for j in range(BB // 16):
        iv = idx_all[hh, pl.ds(16 * j, 16)]
        idxp[s][pl.ds(16 * j, 16)] = lax.shift_right_logical(iv, 1)
        parb[s][pl.ds(16 * j, 16)] = (iv & 1) * D

    def gather_start(s):
      pltpu.async_copy(table_hbm.at[idxp[s]], rows[s], sg[s])

    def gather_wait(s):
      pltpu.make_async_copy(table_hbm.at[idxp[s]], rows[s], sg[s]).wait()

    def store(h, s2):
      return pltpu.make_async_copy(tbuf[s2], out_hbm.at[h, :, w], ss[s2])

    for s in range(NBUF):
      prep_indices(s, s)
      gather_start(s)

    def block_body(p, carry):
      for s in range(NBUF):
        h = NBUF * p + s
        s2 = s % 2
        gather_wait(s)

        if s >= 2:
          store(h - 2, s2).wait()
        else:
          @pl.when(p >= 1)
          def _():
            store(h - 2, s2).wait()

        pars = [parb[s][pl.ds(16 * j, 16)] for j in range(BB // 16)]

        @plsc.parallel_loop(0, 16, unroll=2)
        def _tr(r):
          rot = (lanes + r) & 15
          for k in range(D // 16):
            dvec = rot + 16 * k
            dtv = lax.shift_right_logical(dvec, 3)
            drv = dvec & 7
            for j in range(BB // 16):
              v = plsc.load_gather(rows[s], [bvecs[j], dvec + pars[j]])
              plsc.store_scatter(tbuf[s2], [dtv, drv, bvecs[j]], v)

        @pl.when(p < nblk - 1)
        def _():
          prep_indices(h + NBUF, s)
          gather_start(s)

        store(h, s2).start()
      return carry

    lax.fori_loop(0, nblk, block_body, 0)
    for s2 in range(2):
      store(hist - 2 + s2, s2).wait()

  return gather_kernel


def kernel(token_ids, table):
  b, h = token_ids.shape
  tt = table.T
  vtail = tt.shape[1] - (tt.shape[1] % BB)
  tail_pad = jnp.pad(tt[:, vtail:], ((0, 0), (0, BB - (tt.shape[1] - vtail))))
  pairs = _build_pre(table.shape[0])(tt, tail_pad)
  out5 = _build(b, h)(token_ids.T, pairs)
  return out5.transpose(2, 4, 0, 1, 3).reshape(b, h, D)


# speedup vs baseline: 1.0542x; 1.0542x over previous
import functools

import jax
import jax.numpy as jnp
from jax import lax
from jax.experimental import pallas as pl
from jax.experimental.pallas import tpu as pltpu
from jax.experimental.pallas import tpu_sc as plsc

D = 64
NC = 2
NS = 16
NW = NC * NS
BB = 128
NBUF = 4


@functools.cache
def _build_pre(vocab: int):
  assert vocab == 1000000
  nwin = vocab // BB + 1
  tail = nwin - 1
  nstep = 246
  mesh = plsc.VectorSubcoreMesh(core_axis_name="c", subcore_axis_name="s")

  @functools.partial(
      pl.kernel,
      out_type=jax.ShapeDtypeStruct((vocab // 2, 2 * D), jnp.float32),
      mesh=mesh,
      scratch_types=(
          [pltpu.VMEM((D, BB), jnp.float32) for _ in range(2)]
          + [pltpu.VMEM((D, BB), jnp.float32) for _ in range(2)]
          + [pltpu.SemaphoreType.DMA for _ in range(4)]
      ),
      compiler_params=pltpu.CompilerParams(needs_layout_passes=False),
  )
  def pre_kernel(tt_hbm, tail_hbm, out_hbm, tin0, tin1, tout0, tout1,
                 si0, si1, so0, so1):
    tin = (tin0, tin1)
    tout = (tout0, tout1)
    si = (si0, si1)
    so = (so0, so1)
    w = lax.axis_index("s") * NC + lax.axis_index("c")

    lanes = jnp.arange(16, dtype=jnp.int32)
    l2 = 2 * lanes
    kvecs = [lanes + kk0 for kk0 in range(0, D, 16)]

    def win_of(i):
      raw = w + NW * i
      return jnp.where(raw < nwin, raw, 0)

    def load_in(i, s):
      win = win_of(i)

      @pl.when(win == tail)
      def _():
        pltpu.async_copy(tail_hbm, tin[s], si[s])

      @pl.when(win != tail)
      def _():
        pltpu.async_copy(tt_hbm.at[:, pl.ds(win * BB, BB)], tin[s], si[s])

    def wait_in(s):
      pltpu.make_async_copy(tt_hbm.at[:, pl.ds(0, BB)], tin[s], si[s]).wait()

    def store_out(i, s):
      win = win_of(i)

      @pl.when(win == tail)
      def _():
        dst = out_hbm.at[pl.ds(win * D, D // 2)]
        half = tout[s].at[pl.ds(0, D // 2)]
        pltpu.async_copy(half, dst, so[s])
        pltpu.async_copy(half, dst, so[s])

      @pl.when(win != tail)
      def _():
        pltpu.async_copy(tout[s], out_hbm.at[pl.ds(win * D, D)], so[s])

    def wait_out(s):
      pltpu.make_async_copy(
          tout[s], out_hbm.at[pl.ds(0, D)], so[s]).wait()

    for s in range(2):
      load_in(s, s)

    def step_body(q, carry):
      for s in range(2):
        i = 2 * q + s
        wait_in(s)

        @pl.when(q >= 1)
        def _():
          wait_out(s)

        @plsc.parallel_loop(0, 16, unroll=2)
        def _tr(r):
          rot = (lanes + r) & 15
          for c0 in range(0, 2 * D, 16):
            p = c0 // D
            dvec = rot + (c0 % D)
            cvec = rot + c0
            for ki, kk0 in enumerate(range(0, D, 16)):
              vvec = l2 + (2 * kk0 + p)
              v = plsc.load_gather(tin[s], [dvec, vvec])
              plsc.store_scatter(tout[s], [kvecs[ki], cvec], v)

        store_out(i, s)

        @pl.when(q < nstep // 2 - 1)
        def _():
          load_in(i + 2, s)
      return carry

    lax.fori_loop(0, nstep // 2, step_body, 0)
    for s in range(2):
      wait_out(s)

  return pre_kernel


@functools.cache
def _build(batch: int, hist: int):
  assert batch == NW * BB and hist % NBUF == 0
  nblk = hist // NBUF
  mesh = plsc.VectorSubcoreMesh(core_axis_name="c", subcore_axis_name="s")

  @functools.partial(
      pl.kernel,
      out_type=jax.ShapeDtypeStruct((hist, D // 8, batch // BB, 8, BB),
                                    jnp.float32),
      mesh=mesh,
      scratch_types=[
          pltpu.VMEM((hist, BB), jnp.int32),
      ] + [pltpu.VMEM((BB, 2 * D), jnp.float32) for _ in range(NBUF)]
        + [pltpu.VMEM((D // 8, 8, BB), jnp.float32) for _ in range(2)]
        + [pltpu.VMEM((BB,), jnp.int32) for _ in range(NBUF)]
        + [pltpu.VMEM((BB,), jnp.int32) for _ in range(NBUF)]
        + [pltpu.SemaphoreType.DMA for _ in range(NBUF + 2)],
      compiler_params=pltpu.CompilerParams(needs_layout_passes=False),
  )
  def gather_kernel(tid_hbm, table_hbm, out_hbm, idx_all, *bufs):
    rows = bufs[:NBUF]
    tbuf = bufs[NBUF:NBUF + 2]
    idxp = bufs[NBUF + 2:2 * NBUF + 2]
    parb = bufs[2 * NBUF + 2:3 * NBUF + 2]
    sg = bufs[3 * NBUF + 2:4 * NBUF + 2]
    ss = bufs[4 * NBUF + 2:]
    w = lax.axis_index("s") * NC + lax.axis_index("c")

    pltpu.sync_copy(tid_hbm.at[:, pl.ds(w * BB, BB)], idx_all)

    lanes = jnp.arange(16, dtype=jnp.int32)
    bvecs = [lanes + 16 * j for j in range(BB // 16)]

    def prep_indices(hh, s):
      for j in range(BB // 16):
        iv = idx_all[hh, pl.ds(16 * j, 16)]
        idxp[s][pl.ds(16 * j, 16)] = lax.shift_right_logical(iv, 1)
        parb[s][pl.ds(16 * j, 16)] = (iv & 1) * D

    def gather(s):
      return pltpu.make_async_copy(table_hbm.at[idxp[s]], rows[s], sg[s])

    def store(h, s2):
      return pltpu.make_async_copy(tbuf[s2], out_hbm.at[h, :, w], ss[s2])

    for s in range(NBUF):
      prep_indices(s, s)
      gather(s).start()

    def block_body(p, carry):
      for s in range(NBUF):
        h = NBUF * p + s
        s2 = s % 2
        gather(s).wait()

        if s >= 2:
          store(h - 2, s2).wait()
        else:
          @pl.when(p >= 1)
          def _():
            store(h - 2, s2).wait()

        pars = [parb[s][pl.ds(16 * j, 16)] for j in range(BB // 16)]

        @plsc.parallel_loop(0, 16, unroll=2)
        def _tr(r):
          rot = (lanes + r) & 15
          for k in range(D // 16):
            dvec = rot + 16 * k
            dtv = lax.shift_right_logical(dvec, 3)
            drv = dvec & 7
            for j in range(BB // 16):
              v = plsc.load_gather(rows[s], [bvecs[j], dvec + pars[j]])
              plsc.store_scatter(tbuf[s2], [dtv, drv, bvecs[j]], v)

        @pl.when(p < nblk - 1)
        def _():
          prep_indices(h + NBUF, s)
          gather(s).start()

        store(h, s2).start()
      return carry

    lax.fori_loop(0, nblk, block_body, 0)
    for s2 in range(2):
      store(hist - 2 + s2, s2).wait()

  return gather_kernel


def kernel(token_ids, table):
  b, h = token_ids.shape
  tt = table.T
  vtail = tt.shape[1] - (tt.shape[1] % BB)
  tail_pad = jnp.pad(tt[:, vtail:], ((0, 0), (0, BB - (tt.shape[1] - vtail))))
  pairs = _build_pre(table.shape[0])(tt, tail_pad)
  out5 = _build(b, h)(token_ids.T, pairs)
  return out5.transpose(2, 4, 0, 1, 3).reshape(b, h, D)
